# baseline (speedup 1.0000x reference)
import functools

import jax
import jax.numpy as jnp
from jax import lax
from jax.experimental import pallas as pl
from jax.experimental.pallas import tpu as pltpu
from jax.experimental.pallas import tpu_sc as plsc

N = 10000
E = 320000
D = 128
DEGW = 128
NC = 2
NS = 16
NW = NC * NS
CHUNK = 128
K = 80
KH = K // 2
EP = NW * CHUNK * K
PAD_ROWS = 112
NACC = N + PAD_ROWS
RPT = NACC // NS
DISW = 16


@functools.cache
def _sc_kernels():
    mesh = plsc.VectorSubcoreMesh(core_axis_name="c", subcore_axis_name="s",
                                  num_cores=NC, num_subcores=NS)

    @functools.partial(
        pl.kernel,
        out_type=jax.ShapeDtypeStruct((NC, NACC, DEGW), jnp.float32),
        mesh=mesh,
        scratch_types=[
            pltpu.VMEM((K, CHUNK), jnp.int32),
            pltpu.VMEM((CHUNK, DEGW), jnp.float32),
            pltpu.VMEM_SHARED((NACC, DEGW), jnp.float32),
            pltpu.SemaphoreType.DMA,
        ],
    )
    def deg_kernel(col_hbm, ones_hbm, zeros_hbm, deg_out,
                   col_v, ones_v, acc, sem):
        cid = lax.axis_index("c")
        sid = lax.axis_index("s")
        wid = sid * NC + cid
        pltpu.sync_copy(col_hbm.at[wid], col_v)
        pltpu.sync_copy(ones_hbm, ones_v)
        pltpu.sync_copy(zeros_hbm, acc.at[pl.ds(sid * RPT, RPT)])
        plsc.subcore_barrier()

        def fire(k, carry):
            pltpu.async_copy(ones_v, acc.at[col_v.at[k]], sem, add=True)
            return carry

        lax.fori_loop(0, K, fire, 0)

        def drain(k, carry):
            pltpu.make_async_copy(ones_v, acc.at[col_v.at[k]], sem).wait()
            return carry

        lax.fori_loop(0, K, drain, 0)
        plsc.subcore_barrier()
        pltpu.sync_copy(acc.at[pl.ds(sid * RPT, RPT)],
                        deg_out.at[cid, pl.ds(sid * RPT, RPT)])

    @functools.partial(
        pl.kernel,
        out_type=jax.ShapeDtypeStruct((NC, NACC, D), jnp.float32),
        mesh=mesh,
        scratch_types=[
            pltpu.VMEM((KH, CHUNK), jnp.int32),
            pltpu.VMEM((KH, CHUNK), jnp.int32),
            pltpu.VMEM((CHUNK, D), jnp.float32),
            pltpu.VMEM((CHUNK, D), jnp.float32),
            pltpu.VMEM_SHARED((NACC, D), jnp.float32),
            pltpu.SemaphoreType.DMA,
            pltpu.SemaphoreType.DMA,
        ],
    )
    def msg_kernel(g_hbm, row_hbm, col_hbm, zeros_hbm, out_hbm,
                   row_v, col_v, buf0, buf1, acc, sem0, sem1):
        cid = lax.axis_index("c")
        sid = lax.axis_index("s")
        wid = sid * NC + cid
        pltpu.sync_copy(zeros_hbm, acc.at[pl.ds(sid * RPT, RPT)])
        plsc.subcore_barrier()

        bufs = (buf0, buf1)
        sems = (sem0, sem1)

        HC = CHUNK // 2

        def start_gather(nk, buf, sem):
            pltpu.async_copy(g_hbm.at[row_v.at[nk, pl.ds(0, HC)]],
                             buf.at[pl.ds(0, HC)], sem)
            pltpu.async_copy(g_hbm.at[row_v.at[nk, pl.ds(HC, HC)]],
                             buf.at[pl.ds(HC, HC)], sem)

        def wait_gather(nk, buf, sem):
            pltpu.make_async_copy(g_hbm.at[row_v.at[nk, pl.ds(0, HC)]],
                                  buf.at[pl.ds(0, HC)], sem).wait()
            pltpu.make_async_copy(g_hbm.at[row_v.at[nk, pl.ds(HC, HC)]],
                                  buf.at[pl.ds(HC, HC)], sem).wait()

        def body(j, carry):
            for b in range(2):
                k = j * 2 + b
                nk = lax.rem(k + 1, KH)
                start_gather(nk, bufs[1 - b], sems[1 - b])
                wait_gather(k, bufs[b], sems[b])
                pltpu.sync_copy(bufs[b], acc.at[col_v.at[k]], add=True)
            return carry

        for h in range(2):
            pltpu.sync_copy(row_hbm.at[wid, pl.ds(h * KH, KH)], row_v)
            pltpu.sync_copy(col_hbm.at[wid, pl.ds(h * KH, KH)], col_v)
            start_gather(0, buf0, sem0)
            lax.fori_loop(0, KH // 2, body, 0)
            wait_gather(0, buf0, sem0)

        plsc.subcore_barrier()
        pltpu.sync_copy(acc.at[pl.ds(sid * RPT, RPT)],
                        out_hbm.at[cid, pl.ds(sid * RPT, RPT)])

    return deg_kernel, msg_kernel


def _k1_body(degs_ref, x_ref, w1_ref, b1_ref, g_ref, dis_ref):
    deg = degs_ref[0, :N, :] + degs_ref[1, :N, :] + 1.0
    dis = lax.rsqrt(deg)
    dis_ref[...] = dis[:, :DISW]
    h = lax.dot_general(x_ref[...], w1_ref[...], (((1,), (1,)), ((), ())),
                        preferred_element_type=jnp.float32)
    g_ref[...] = dis[:, :1] * (h + b1_ref[...])


def _k2_body(s_ref, g1_ref, dis_ref, gamma_ref, beta_ref, w2_ref, b2_ref,
             g2_ref):
    dis = dis_ref[...][:, :1]
    out1 = dis * (s_ref[0, :N, :] + s_ref[1, :N, :] + g1_ref[...])
    mu = jnp.mean(out1, axis=0, keepdims=True)
    var = jnp.mean(out1 * out1, axis=0, keepdims=True) - mu * mu
    y = gamma_ref[...] * (out1 - mu) * lax.rsqrt(var + 1e-5) + beta_ref[...]
    y = jnp.where(y >= 0, y, 0.1 * y)
    h2 = lax.dot_general(y, w2_ref[...], (((1,), (1,)), ((), ())),
                         preferred_element_type=jnp.float32)
    g2_ref[...] = dis * (h2 + b2_ref[...])


def _k3_body(s_ref, g2_ref, dis_ref, gamma_ref, beta_ref, y_ref):
    dis = dis_ref[...][:, :1]
    out2 = dis * (s_ref[0, :N, :] + s_ref[1, :N, :] + g2_ref[...])
    mu = jnp.mean(out2, axis=0, keepdims=True)
    var = jnp.mean(out2 * out2, axis=0, keepdims=True) - mu * mu
    y = gamma_ref[...] * (out2 - mu) * lax.rsqrt(var + 1e-5) + beta_ref[...]
    y_ref[...] = jnp.where(y >= 0, y, 0.1 * y)


_k1 = pl.pallas_call(
    _k1_body,
    out_shape=(jax.ShapeDtypeStruct((N, D), jnp.float32),
               jax.ShapeDtypeStruct((N, DISW), jnp.float32)),
)
_k2 = pl.pallas_call(
    _k2_body,
    out_shape=jax.ShapeDtypeStruct((N, D), jnp.float32),
)
_k3 = pl.pallas_call(
    _k3_body,
    out_shape=jax.ShapeDtypeStruct((N, D), jnp.float32),
)


def kernel(x, edge_index, W1, b1, gamma1, beta1, W2, b2, gamma2, beta2):
    row = edge_index[0]
    col = edge_index[1]
    pad = EP - E
    pad_gather = (jnp.arange(pad, dtype=jnp.int32) * 37) % N
    pad_scatter = N + (jnp.arange(pad, dtype=jnp.int32) % PAD_ROWS)
    row_p = jnp.concatenate([row, pad_gather]).reshape(NW, K, CHUNK)
    col_p = jnp.concatenate([col, pad_scatter]).reshape(NW, K, CHUNK)
    ones_blk = jnp.ones((CHUNK, DEGW), jnp.float32)
    zeros_blk = jnp.zeros((RPT, D), jnp.float32)

    _deg_kernel, _msg_kernel = _sc_kernels()
    degs = _deg_kernel(col_p, ones_blk, zeros_blk)
    g1, dis16 = _k1(degs, x, W1, b1.reshape(1, D))
    s1 = _msg_kernel(g1, row_p, col_p, zeros_blk)
    g2 = _k2(s1, g1, dis16, gamma1.reshape(1, D), beta1.reshape(1, D),
             W2, b2.reshape(1, D))
    s2 = _msg_kernel(g2, row_p, col_p, zeros_blk)
    y = _k3(s2, g2, dis16, gamma2.reshape(1, D), beta2.reshape(1, D))
    return y


# speedup vs baseline: 1.1107x; 1.1107x over previous
import functools

import jax
import jax.numpy as jnp
from jax import lax
from jax.experimental import pallas as pl
from jax.experimental.pallas import tpu as pltpu
from jax.experimental.pallas import tpu_sc as plsc

N = 10000
E = 320000
D = 128
NC = 2
NS = 16
NW = NC * NS
CHUNK = 128
K = 80
KH = K // 2
EW = K * CHUNK
EP = NW * EW
PAD_ROWS = 112
NACC = N + PAD_ROWS
RPT = NACC // NS
NB = 8
HW = NACC * NB
HR = HW // 128


@functools.cache
def _sc_kernels():
    mesh = plsc.VectorSubcoreMesh(core_axis_name="c", subcore_axis_name="s",
                                  num_cores=NC, num_subcores=NS)

    @functools.partial(
        pl.kernel,
        out_type=jax.ShapeDtypeStruct((NW, HW), jnp.float32),
        mesh=mesh,
        compiler_params=pltpu.CompilerParams(needs_layout_passes=False),
        scratch_types=[
            pltpu.VMEM((EW,), jnp.int32),
            pltpu.VMEM((HW,), jnp.float32),
        ],
    )
    def hist_kernel(col_hbm, zeros_hbm, hist_out, col_v, hist_v):
        cid = lax.axis_index("c")
        sid = lax.axis_index("s")
        wid = sid * NC + cid
        pltpu.sync_copy(col_hbm.at[wid], col_v)
        pltpu.sync_copy(zeros_hbm, hist_v)
        bank = lax.iota(jnp.int32, 16) & 7
        ones = jnp.ones((16,), jnp.float32)

        def body(i, carry):
            idx = col_v[pl.ds(i * 16, 16)]
            plsc.addupdate_scatter(hist_v, [idx * NB + bank], ones)
            return carry

        lax.fori_loop(0, EW // 16, body, 0)
        pltpu.sync_copy(hist_v, hist_out.at[wid])

    @functools.partial(
        pl.kernel,
        out_type=jax.ShapeDtypeStruct((NC, NACC, D), jnp.float32),
        mesh=mesh,
        scratch_types=[
            pltpu.VMEM((KH, CHUNK), jnp.int32),
            pltpu.VMEM((KH, CHUNK), jnp.int32),
            pltpu.VMEM((CHUNK, D), jnp.float32),
            pltpu.VMEM((CHUNK, D), jnp.float32),
            pltpu.VMEM_SHARED((NACC, D), jnp.float32),
            pltpu.SemaphoreType.DMA,
            pltpu.SemaphoreType.DMA,
        ],
    )
    def msg_kernel(g_hbm, row_hbm, col_hbm, zeros_hbm, out_hbm,
                   row_v, col_v, buf0, buf1, acc, sem0, sem1):
        cid = lax.axis_index("c")
        sid = lax.axis_index("s")
        wid = sid * NC + cid
        pltpu.sync_copy(zeros_hbm, acc.at[pl.ds(sid * RPT, RPT)])
        plsc.subcore_barrier()

        bufs = (buf0, buf1)
        sems = (sem0, sem1)

        def body(j, carry):
            for b in range(2):
                k = j * 2 + b
                nk = lax.rem(k + 1, KH)
                pltpu.async_copy(g_hbm.at[row_v.at[nk]],
                                 bufs[1 - b], sems[1 - b])
                pltpu.make_async_copy(g_hbm.at[row_v.at[k]],
                                      bufs[b], sems[b]).wait()
                pltpu.sync_copy(bufs[b], acc.at[col_v.at[k]], add=True)
            return carry

        for h in range(2):
            pltpu.sync_copy(row_hbm.at[wid, pl.ds(h * KH, KH)], row_v)
            pltpu.sync_copy(col_hbm.at[wid, pl.ds(h * KH, KH)], col_v)
            pltpu.async_copy(g_hbm.at[row_v.at[0]], buf0, sem0)
            lax.fori_loop(0, KH // 2, body, 0)
            pltpu.make_async_copy(g_hbm.at[row_v.at[0]], buf0, sem0).wait()

        plsc.subcore_barrier()
        pltpu.sync_copy(acc.at[pl.ds(sid * RPT, RPT)],
                        out_hbm.at[cid, pl.ds(sid * RPT, RPT)])

    return hist_kernel, msg_kernel


def _merge_body(h_ref, gmat_ref, dis_ref):
    s = jnp.sum(h_ref[...], axis=0)
    degb = lax.dot_general(s, gmat_ref[...], (((1,), (0,)), ((), ())),
                           preferred_element_type=jnp.float32)
    dis_ref[...] = lax.rsqrt(degb + 1.0)


def _k1_body(dis_ref, x_ref, w1_ref, b1_ref, g_ref):
    dis = dis_ref[:N, :]
    h = lax.dot_general(x_ref[...], w1_ref[...], (((1,), (1,)), ((), ())),
                        preferred_element_type=jnp.float32)
    g_ref[...] = dis * (h + b1_ref[...])


def _k2_body(s_ref, g1_ref, dis_ref, gamma_ref, beta_ref, w2_ref, b2_ref,
             g2_ref):
    dis = dis_ref[:N, :]
    out1 = dis * (s_ref[0, :N, :] + s_ref[1, :N, :] + g1_ref[...])
    mu = jnp.mean(out1, axis=0, keepdims=True)
    var = jnp.mean(out1 * out1, axis=0, keepdims=True) - mu * mu
    y = gamma_ref[...] * (out1 - mu) * lax.rsqrt(var + 1e-5) + beta_ref[...]
    y = jnp.where(y >= 0, y, 0.1 * y)
    h2 = lax.dot_general(y, w2_ref[...], (((1,), (1,)), ((), ())),
                         preferred_element_type=jnp.float32)
    g2_ref[...] = dis * (h2 + b2_ref[...])


def _k3_body(s_ref, g2_ref, dis_ref, gamma_ref, beta_ref, y_ref):
    dis = dis_ref[:N, :]
    out2 = dis * (s_ref[0, :N, :] + s_ref[1, :N, :] + g2_ref[...])
    mu = jnp.mean(out2, axis=0, keepdims=True)
    var = jnp.mean(out2 * out2, axis=0, keepdims=True) - mu * mu
    y = gamma_ref[...] * (out2 - mu) * lax.rsqrt(var + 1e-5) + beta_ref[...]
    y_ref[...] = jnp.where(y >= 0, y, 0.1 * y)


_merge = pl.pallas_call(
    _merge_body,
    out_shape=jax.ShapeDtypeStruct((HR, 16), jnp.float32),
)
_k1 = pl.pallas_call(
    _k1_body,
    out_shape=jax.ShapeDtypeStruct((N, D), jnp.float32),
)
_k2 = pl.pallas_call(
    _k2_body,
    out_shape=jax.ShapeDtypeStruct((N, D), jnp.float32),
)
_k3 = pl.pallas_call(
    _k3_body,
    out_shape=jax.ShapeDtypeStruct((N, D), jnp.float32),
)


def kernel(x, edge_index, W1, b1, gamma1, beta1, W2, b2, gamma2, beta2):
    row = edge_index[0]
    col = edge_index[1]
    pad = EP - E
    pad_gather = (jnp.arange(pad, dtype=jnp.int32) * 37) % N
    pad_scatter = N + (jnp.arange(pad, dtype=jnp.int32) % PAD_ROWS)
    row_p = jnp.concatenate([row, pad_gather]).reshape(NW, K, CHUNK)
    col_p = jnp.concatenate([col, pad_scatter]).reshape(NW, K, CHUNK)
    zeros_blk = jnp.zeros((RPT, D), jnp.float32)
    zeros_hist = jnp.zeros((HW,), jnp.float32)
    gmat = jnp.repeat(jnp.eye(16, dtype=jnp.float32), NB, axis=0)

    _hist_kernel, _msg_kernel = _sc_kernels()
    hists = _hist_kernel(col_p.reshape(NW, EW), zeros_hist)
    dis_blk = _merge(hists.reshape(NW, HR, 128), gmat)
    dis_col = dis_blk.reshape(NACC, 1)
    g1 = _k1(dis_col, x, W1, b1.reshape(1, D))
    s1 = _msg_kernel(g1, row_p, col_p, zeros_blk)
    g2 = _k2(s1, g1, dis_col, gamma1.reshape(1, D), beta1.reshape(1, D),
             W2, b2.reshape(1, D))
    s2 = _msg_kernel(g2, row_p, col_p, zeros_blk)
    y = _k3(s2, g2, dis_col, gamma2.reshape(1, D), beta2.reshape(1, D))
    return y


# speedup vs baseline: 1.1478x; 1.0334x over previous
import functools

import jax
import jax.numpy as jnp
from jax import lax
from jax.experimental import pallas as pl
from jax.experimental.pallas import tpu as pltpu
from jax.experimental.pallas import tpu_sc as plsc

N = 10000
E = 320000
D = 128
NC = 2
NS = 16
NW = NC * NS
CHUNK = 128
K = 80
KH = K // 2
EW = K * CHUNK
EP = NW * EW
PAD_ROWS = 112
NACC = N + PAD_ROWS
RPT = NACC // NS
NB = 8
HW = NACC * NB
HR = HW // 128


@functools.cache
def _sc_kernels():
    mesh = plsc.VectorSubcoreMesh(core_axis_name="c", subcore_axis_name="s",
                                  num_cores=NC, num_subcores=NS)

    @functools.partial(
        pl.kernel,
        out_type=jax.ShapeDtypeStruct((NW, HR, 128), jnp.float32),
        mesh=mesh,
        compiler_params=pltpu.CompilerParams(needs_layout_passes=False),
        scratch_types=[
            pltpu.VMEM((EW,), jnp.int32),
            pltpu.VMEM((HR, 128), jnp.float32),
        ],
    )
    def hist_kernel(col_hbm, zeros_hbm, hist_out, col_v, hist_v):
        cid = lax.axis_index("c")
        sid = lax.axis_index("s")
        wid = sid * NC + cid
        pltpu.sync_copy(col_hbm.at[wid], col_v)
        pltpu.sync_copy(zeros_hbm, hist_v)
        bank = lax.iota(jnp.int32, 16) & 7
        ones = jnp.ones((16,), jnp.float32)

        def body(i, carry):
            idx = col_v[pl.ds(i * 16, 16)]
            addr = idx * NB + bank
            plsc.addupdate_scatter(hist_v, [addr >> 7, addr & 127], ones)
            return carry

        lax.fori_loop(0, EW // 16, body, 0)
        pltpu.sync_copy(hist_v, hist_out.at[wid])

    @functools.partial(
        pl.kernel,
        out_type=jax.ShapeDtypeStruct((NC, NACC, D), jnp.float32),
        mesh=mesh,
        scratch_types=[
            pltpu.VMEM((KH, CHUNK), jnp.int32),
            pltpu.VMEM((KH, CHUNK), jnp.int32),
            pltpu.VMEM((CHUNK, D), jnp.float32),
            pltpu.VMEM((CHUNK, D), jnp.float32),
            pltpu.VMEM_SHARED((NACC, D), jnp.float32),
            pltpu.SemaphoreType.DMA,
            pltpu.SemaphoreType.DMA,
        ],
    )
    def msg_kernel(g_hbm, row_hbm, col_hbm, zeros_hbm, out_hbm,
                   row_v, col_v, buf0, buf1, acc, sem0, sem1):
        cid = lax.axis_index("c")
        sid = lax.axis_index("s")
        wid = sid * NC + cid
        pltpu.sync_copy(zeros_hbm, acc.at[pl.ds(sid * RPT, RPT)])
        plsc.subcore_barrier()

        bufs = (buf0, buf1)
        sems = (sem0, sem1)

        def body(j, carry):
            for b in range(2):
                k = j * 2 + b
                nk = lax.rem(k + 1, KH)
                pltpu.async_copy(g_hbm.at[row_v.at[nk]],
                                 bufs[1 - b], sems[1 - b])
                pltpu.make_async_copy(g_hbm.at[row_v.at[k]],
                                      bufs[b], sems[b]).wait()
                pltpu.sync_copy(bufs[b], acc.at[col_v.at[k]], add=True)
            return carry

        for h in range(2):
            pltpu.sync_copy(row_hbm.at[wid, pl.ds(h * KH, KH)], row_v)
            pltpu.sync_copy(col_hbm.at[wid, pl.ds(h * KH, KH)], col_v)
            pltpu.async_copy(g_hbm.at[row_v.at[0]], buf0, sem0)
            lax.fori_loop(0, KH // 2, body, 0)
            pltpu.make_async_copy(g_hbm.at[row_v.at[0]], buf0, sem0).wait()

        plsc.subcore_barrier()
        pltpu.sync_copy(acc.at[pl.ds(sid * RPT, RPT)],
                        out_hbm.at[cid, pl.ds(sid * RPT, RPT)])

    return hist_kernel, msg_kernel


def _merge_body(h_ref, gmat_ref, dis_ref):
    s = jnp.sum(h_ref[...], axis=0)
    degb = lax.dot_general(s, gmat_ref[...], (((1,), (0,)), ((), ())),
                           preferred_element_type=jnp.float32)
    dis_ref[...] = lax.rsqrt(degb + 1.0)


def _k1_body(dis_ref, x_ref, w1_ref, b1_ref, g_ref):
    dis = dis_ref[:N, :]
    h = lax.dot_general(x_ref[...], w1_ref[...], (((1,), (1,)), ((), ())),
                        preferred_element_type=jnp.float32)
    g_ref[...] = dis * (h + b1_ref[...])


def _k2_body(s_ref, g1_ref, dis_ref, gamma_ref, beta_ref, w2_ref, b2_ref,
             g2_ref):
    dis = dis_ref[:N, :]
    out1 = dis * (s_ref[0, :N, :] + s_ref[1, :N, :] + g1_ref[...])
    mu = jnp.mean(out1, axis=0, keepdims=True)
    var = jnp.mean(out1 * out1, axis=0, keepdims=True) - mu * mu
    y = gamma_ref[...] * (out1 - mu) * lax.rsqrt(var + 1e-5) + beta_ref[...]
    y = jnp.where(y >= 0, y, 0.1 * y)
    h2 = lax.dot_general(y, w2_ref[...], (((1,), (1,)), ((), ())),
                         preferred_element_type=jnp.float32)
    g2_ref[...] = dis * (h2 + b2_ref[...])


def _k3_body(s_ref, g2_ref, dis_ref, gamma_ref, beta_ref, y_ref):
    dis = dis_ref[:N, :]
    out2 = dis * (s_ref[0, :N, :] + s_ref[1, :N, :] + g2_ref[...])
    mu = jnp.mean(out2, axis=0, keepdims=True)
    var = jnp.mean(out2 * out2, axis=0, keepdims=True) - mu * mu
    y = gamma_ref[...] * (out2 - mu) * lax.rsqrt(var + 1e-5) + beta_ref[...]
    y_ref[...] = jnp.where(y >= 0, y, 0.1 * y)


_merge = pl.pallas_call(
    _merge_body,
    out_shape=jax.ShapeDtypeStruct((HR, 16), jnp.float32),
)
_k1 = pl.pallas_call(
    _k1_body,
    out_shape=jax.ShapeDtypeStruct((N, D), jnp.float32),
)
_k2 = pl.pallas_call(
    _k2_body,
    out_shape=jax.ShapeDtypeStruct((N, D), jnp.float32),
)
_k3 = pl.pallas_call(
    _k3_body,
    out_shape=jax.ShapeDtypeStruct((N, D), jnp.float32),
)


def kernel(x, edge_index, W1, b1, gamma1, beta1, W2, b2, gamma2, beta2):
    row = edge_index[0]
    col = edge_index[1]
    pad = EP - E
    pad_gather = (jnp.arange(pad, dtype=jnp.int32) * 37) % N
    pad_scatter = N + (jnp.arange(pad, dtype=jnp.int32) % PAD_ROWS)
    row_p = jnp.concatenate([row, pad_gather]).reshape(NW, K, CHUNK)
    col_p = jnp.concatenate([col, pad_scatter]).reshape(NW, K, CHUNK)
    zeros_blk = jnp.zeros((RPT, D), jnp.float32)
    zeros_hist = jnp.zeros((HR, 128), jnp.float32)
    gmat = jnp.repeat(jnp.eye(16, dtype=jnp.float32), NB, axis=0)

    _hist_kernel, _msg_kernel = _sc_kernels()
    hists = _hist_kernel(col_p.reshape(NW, EW), zeros_hist)
    dis_blk = _merge(hists, gmat)
    dis_col = dis_blk.reshape(NACC, 1)
    g1 = _k1(dis_col, x, W1, b1.reshape(1, D))
    s1 = _msg_kernel(g1, row_p, col_p, zeros_blk)
    g2 = _k2(s1, g1, dis_col, gamma1.reshape(1, D), beta1.reshape(1, D),
             W2, b2.reshape(1, D))
    s2 = _msg_kernel(g2, row_p, col_p, zeros_blk)
    y = _k3(s2, g2, dis_col, gamma2.reshape(1, D), beta2.reshape(1, D))
    return y
